# baseline (speedup 1.0000x reference)
import jax
import jax.numpy as jnp
from jax import lax
from jax.experimental import pallas as pl
from jax.experimental.pallas import tpu as pltpu
from jax.experimental.pallas import tpu_sc as plsc

N = 1024
U = 200
D = 64
VECS = D // 16
UP = U // 2

_info = plsc.get_sparse_core_info()
NC, NS = _info.num_cores, _info.num_subcores
NW = NC * NS
SEQ_PER_W = N // NW
PAIRS_PER_W = SEQ_PER_W // 2

CH0 = 128
CH1 = U - CH0


def _sc_body(y2_hbm, half_hbm, table_hbm, w_hbm, out_hbm,
             idx_v, half_v, rows_v, outb_v, w_v, sem):
    wid = lax.axis_index("s") * NC + lax.axis_index("c")

    pltpu.sync_copy(w_hbm, w_v)
    w0 = [w_v[0, pl.ds(16 * j, 16)] for j in range(VECS)]
    w1 = [w_v[1, pl.ds(16 * j, 16)] for j in range(VECS)]
    zero = jnp.zeros((16,), jnp.float32)

    def pair_body(p_i, carry):
        pair_base = (wid * PAIRS_PER_W + p_i) * 2 * U

        def do_seq(half_idx, out_off):
            base = pair_base + half_idx * U
            pltpu.sync_copy(y2_hbm.at[pl.ds(base, U)], idx_v)
            pltpu.sync_copy(half_hbm.at[pl.ds(base, U)], half_v.at[pl.ds(0, U)])
            cp0 = pltpu.async_copy(
                table_hbm.at[idx_v.at[pl.ds(0, CH0)]],
                rows_v.at[pl.ds(0, CH0)], sem)
            cp1 = pltpu.async_copy(
                table_hbm.at[idx_v.at[pl.ds(CH0, CH1)]],
                rows_v.at[pl.ds(CH0, CH1)], sem)
            cp0.wait()
            cp1.wait()

            def blk_body(blk, prev):
                rbase = 8 * blk
                hv = half_v[pl.ds(rbase, 16)] * D
                for t in range(4):
                    ua = rbase + 2 * t
                    offa = hv[2 * t]
                    offb = hv[2 * t + 1]
                    cura = tuple(
                        rows_v[ua, pl.ds(offa + 16 * j, 16)]
                        for j in range(VECS))
                    curb = tuple(
                        rows_v[ua + 1, pl.ds(offb + 16 * j, 16)]
                        for j in range(VECS))
                    orow = out_off + 4 * blk + t
                    for j in range(VECS):
                        outb_v[orow, pl.ds(16 * j, 16)] = jnp.maximum(
                            prev[j] * w0[j] + cura[j] * w1[j], 0.0)
                        outb_v[orow, pl.ds(D + 16 * j, 16)] = jnp.maximum(
                            cura[j] * w0[j] + curb[j] * w1[j], 0.0)
                    prev = curb
                return prev

            lax.fori_loop(0, UP // 4, blk_body, (zero,) * VECS)

        do_seq(0, 0)
        do_seq(1, UP)
        out_base = pl.multiple_of(pair_base // 2, 8)
        pltpu.sync_copy(outb_v, out_hbm.at[pl.ds(out_base, U)])
        return carry

    lax.fori_loop(0, PAIRS_PER_W, pair_body, 0)


_sc_call = pl.kernel(
    _sc_body,
    out_type=jax.ShapeDtypeStruct((N * U // 2, 2 * D), jnp.float32),
    mesh=plsc.VectorSubcoreMesh(core_axis_name="c", subcore_axis_name="s"),
    scratch_types=[
        pltpu.VMEM((U,), jnp.int32),
        pltpu.VMEM((U + 16, ), jnp.int32),
        pltpu.VMEM((U, 2 * D), jnp.float32),
        pltpu.VMEM((U, 2 * D), jnp.float32),
        pltpu.VMEM((2, D), jnp.float32),
        pltpu.SemaphoreType.DMA,
    ],
)


@jax.jit
def kernel(y, table, conv_w):
    y_flat = y.reshape(N * U).astype(jnp.int32)
    y2 = y_flat >> 1
    half = y_flat & 1
    t2 = table.reshape(table.shape[0] // 2, 2 * D)
    w = conv_w.T
    out = _sc_call(y2, half, t2, w)
    return out.reshape(N, U, D)


# speedup vs baseline: 1.0302x; 1.0302x over previous
import jax
import jax.numpy as jnp
from jax import lax
from jax.experimental import pallas as pl
from jax.experimental.pallas import tpu as pltpu
from jax.experimental.pallas import tpu_sc as plsc

N = 1024
U = 200
D = 64
VECS = D // 16

_info = plsc.get_sparse_core_info()
NC, NS = _info.num_cores, _info.num_subcores
NW = NC * NS
SEQ_PER_W = N // NW
PAIRS_PER_W = SEQ_PER_W // 2

CH0 = 128
CH1 = U - CH0


def _sc_body(y2_hbm, half_hbm, table_hbm, w_hbm, out_hbm,
             idx_v, half_v, rows_v, outb_v, w_v, sem):
    wid = lax.axis_index("s") * NC + lax.axis_index("c")

    pltpu.sync_copy(w_hbm, w_v)
    w0 = [w_v[0, pl.ds(16 * j, 16)] for j in range(VECS)]
    w1 = [w_v[1, pl.ds(16 * j, 16)] for j in range(VECS)]
    zero = jnp.zeros((16,), jnp.float32)

    def pair_body(p_i, carry):
        pair0 = wid * PAIRS_PER_W + p_i
        pair_base = pair0 * 2 * U

        def do_seq(si):
            base = pair_base + si * U
            pltpu.sync_copy(y2_hbm.at[pl.ds(base, U)], idx_v)
            pltpu.sync_copy(half_hbm.at[pl.ds(base, U)],
                            half_v.at[pl.ds(0, U)])
            cp0 = pltpu.async_copy(
                table_hbm.at[idx_v.at[pl.ds(0, CH0)]],
                rows_v.at[pl.ds(0, CH0)], sem)
            cp1 = pltpu.async_copy(
                table_hbm.at[idx_v.at[pl.ds(CH0, CH1)]],
                rows_v.at[pl.ds(CH0, CH1)], sem)
            cp0.wait()
            cp1.wait()

            def blk_body(blk, prev):
                rbase = 8 * blk
                hv = half_v[pl.ds(rbase, 16)] * D
                for t in range(8):
                    u = rbase + t
                    off = hv[t]
                    cur = tuple(
                        rows_v[u, pl.ds(off + 16 * j, 16)]
                        for j in range(VECS))
                    for j in range(VECS):
                        outb_v[si, u, pl.ds(16 * j, 16)] = jnp.maximum(
                            prev[j] * w0[j] + cur[j] * w1[j], 0.0)
                    prev = cur
                return prev

            lax.fori_loop(0, U // 8, blk_body, (zero,) * VECS)

        do_seq(0)
        do_seq(1)
        pltpu.sync_copy(outb_v, out_hbm.at[pl.ds(pair0 * 2, 2)])
        return carry

    lax.fori_loop(0, PAIRS_PER_W, pair_body, 0)


_sc_call = pl.kernel(
    _sc_body,
    out_type=jax.ShapeDtypeStruct((N, U, D), jnp.float32),
    mesh=plsc.VectorSubcoreMesh(core_axis_name="c", subcore_axis_name="s"),
    scratch_types=[
        pltpu.VMEM((U,), jnp.int32),
        pltpu.VMEM((U + 16,), jnp.int32),
        pltpu.VMEM((U, 2 * D), jnp.float32),
        pltpu.VMEM((2, U, D), jnp.float32),
        pltpu.VMEM((2, D), jnp.float32),
        pltpu.SemaphoreType.DMA,
    ],
)


@jax.jit
def kernel(y, table, conv_w):
    y_flat = y.reshape(N * U).astype(jnp.int32)
    y2 = y_flat >> 1
    half = y_flat & 1
    t2 = table.reshape(table.shape[0] // 2, 2 * D)
    w = conv_w.T
    return _sc_call(y2, half, t2, w)
